# baseline (speedup 1.0000x reference)
import jax
import jax.numpy as jnp
from jax import lax
from jax.experimental import pallas as pl
from jax.experimental.pallas import tpu as pltpu, tpu_sc as plsc

B, T, D = 4, 8192, 1024
NW = 32
T_PER_W = T // NW
CHUNK_T = 8
N_CHUNK = T_PER_W // CHUNK_T
N_STEP = N_CHUNK * B


def _sc_body(x_hbm, emb_hbm, out_hbm, ebuf0, ebuf1,
             ibuf0, ibuf1, ibuf2, ibuf3, ibuf4, ibuf5, ibuf6, ibuf7,
             obuf0, obuf1, obuf2, obuf3,
             lsem0, lsem1, lsem2, lsem3, lsem4, lsem5, lsem6, lsem7,
             ssem0, ssem1, ssem2, ssem3, esem0, esem1):
    cid = lax.axis_index("c")
    sid = lax.axis_index("s")
    wid = sid * 2 + cid
    t_base = wid * T_PER_W

    ebufs = [ebuf0, ebuf1]
    esems = [esem0, esem1]
    ibufs = [ibuf0, ibuf1, ibuf2, ibuf3, ibuf4, ibuf5, ibuf6, ibuf7]
    lsems = [lsem0, lsem1, lsem2, lsem3, lsem4, lsem5, lsem6, lsem7]
    obufs = [obuf0, obuf1, obuf2, obuf3]
    ssems = [ssem0, ssem1, ssem2, ssem3]

    def c_t0(chunk):
        return t_base + chunk * CHUNK_T

    def start_load(chunk, b, islot):
        pltpu.async_copy(x_hbm.at[b, pl.ds(c_t0(chunk), CHUNK_T)],
                         ibufs[islot], lsems[islot])

    def start_emb(chunk, eslot):
        pltpu.async_copy(emb_hbm.at[pl.ds(c_t0(chunk), CHUNK_T)],
                         ebufs[eslot], esems[eslot])

    start_emb(0, 0)
    start_emb(1, 1)
    for s in range(8):
        start_load(s // 4, s % 4, s)

    def group_body(p, _):
        for s in range(8):
            chunk = p * 2 + s // 4
            b = s % 4
            ib = ibufs[s]
            ob = obufs[b]
            eb = ebufs[s // 4]

            pltpu.make_async_copy(
                x_hbm.at[b, pl.ds(c_t0(chunk), CHUNK_T)], ib,
                lsems[s]).wait()

            if b == 0:
                pltpu.make_async_copy(
                    emb_hbm.at[pl.ds(c_t0(chunk), CHUNK_T)], eb,
                    esems[s // 4]).wait()

            @pl.when(chunk >= 1)
            def _():
                pltpu.make_async_copy(
                    ob, out_hbm.at[b, pl.ds(c_t0(chunk - 1), CHUNK_T)],
                    ssems[b]).wait()

            for r in range(CHUNK_T):
                @plsc.parallel_loop(0, D, 16, unroll=8)
                def _add(o):
                    ob[r, pl.ds(o, 16)] = (ib[r, pl.ds(o, 16)] +
                                           eb[r, pl.ds(o, 16)])

            pltpu.async_copy(ob, out_hbm.at[b, pl.ds(c_t0(chunk), CHUNK_T)],
                             ssems[b])

            @pl.when(chunk + 2 < N_CHUNK)
            def _():
                start_load(chunk + 2, b, s)

            if b == B - 1:
                @pl.when(chunk + 2 < N_CHUNK)
                def _():
                    start_emb(chunk + 2, s // 4)

        return 0

    lax.fori_loop(0, N_CHUNK // 2, group_body, 0)

    for b in range(B):
        pltpu.make_async_copy(obufs[b],
                              out_hbm.at[b, pl.ds(c_t0(N_CHUNK - 1), CHUNK_T)],
                              ssems[b]).wait()


def kernel(x, emb_table):
    assert x.shape == (B, T, D) and emb_table.shape == (T, D)
    mesh = plsc.VectorSubcoreMesh(core_axis_name="c", subcore_axis_name="s")
    vm = lambda: pltpu.VMEM((CHUNK_T, D), jnp.float32)
    sem = lambda: pltpu.SemaphoreType.DMA
    return pl.kernel(
        _sc_body,
        mesh=mesh,
        out_type=jax.ShapeDtypeStruct((B, T, D), jnp.float32),
        scratch_types=[vm(), vm(),
                       vm(), vm(), vm(), vm(), vm(), vm(), vm(), vm(),
                       vm(), vm(), vm(), vm(),
                       sem(), sem(), sem(), sem(), sem(), sem(), sem(),
                       sem(), sem(), sem(), sem(), sem(), sem(), sem()],
    )(x, emb_table)
